# baseline (speedup 1.0000x reference)
import jax
import jax.numpy as jnp
from jax import lax
from jax.experimental import pallas as pl
from jax.experimental.pallas import tpu as pltpu
from jax.experimental.pallas import tpu_sc as plsc

_NUM_BUCKETS = 32
_H = 16
_Q = 2048
_K = 2048
_LINE = 4224
_NS = 16


def _line_body(scal_ref, table_ref, out_ref):
    h = pl.program_id(0)
    delta = scal_ref[0]
    boff = scal_ref[1]
    u = jax.lax.broadcasted_iota(jnp.int32, (1, _LINE), 1)
    rel = (2047 - u) + delta
    neg16 = jnp.where(rel < 0, 16, 0)
    n = jnp.abs(rel)
    nf = n.astype(jnp.float32)
    val_large = 8 + (jnp.log(nf / 8.0) / jnp.log(16.0) * 8.0).astype(jnp.int32)
    val_large = jnp.minimum(val_large, 15)
    bucket = neg16 + jnp.where(n < 8, n, val_large) + boff
    idx = jnp.mod(bucket, _NUM_BUCKETS)
    line = jnp.zeros((1, _LINE), jnp.float32)
    for b in range(_NUM_BUCKETS):
        line = jnp.where(idx == b, table_ref[h, b], line)
    for t in range(_NS):
        row = line if t == 0 else jnp.concatenate(
            [jnp.zeros((1, t), jnp.float32), line[:, : _LINE - t]], axis=1)
        out_ref[0, pl.ds(t, 1), :] = row


def _sc_expand_body(w16_hbm, out_hbm, stk, sem):
    c = lax.axis_index("c")
    s = lax.axis_index("s")
    wid = s * 2 + c
    h = wid // 2
    half = wid % 2
    rbase = half * (_Q // 2)
    pltpu.sync_copy(w16_hbm.at[h], stk)

    NRUN = 64

    def fire_run(m):
        i0 = rbase + 15 + 16 * m
        o16 = pl.multiple_of(2047 - i0, 16)
        r0 = h * _Q + i0
        return pltpu.async_copy(
            stk.at[:, pl.ds(o16, _K)],
            out_hbm.at[pl.ds(r0, 16), :], sem)

    def body(p, carry):
        ha = fire_run(2 * p)
        hb = fire_run(2 * p + 1)
        ha.wait()
        hb.wait()
        return carry

    lax.fori_loop(0, NRUN // 2, body, 0)

    def fire_row(i):
        t = lax.rem(i + 1, _NS)
        o16 = (2047 - i) + t
        src_off = pl.multiple_of(t * _LINE + o16, 16)
        return pltpu.async_copy(
            stk.at[t, pl.ds(o16, _K)],
            out_hbm.at[h * _Q + i, :], sem)

    @pl.when(half == 0)
    def _low_rows():
        hs = [fire_row(i) for i in range(15)]
        for hd in hs:
            hd.wait()

    @pl.when(half == 1)
    def _last_row():
        fire_row(2047).wait()


def kernel(q_len, k_len, bidirectional, relative_attention_bias):
    delta = jnp.asarray(q_len, jnp.int32) - jnp.asarray(k_len, jnp.int32)
    boff = jnp.asarray(bidirectional, jnp.int32) - 1
    scal = jnp.stack([delta, boff])
    w16 = pl.pallas_call(
        _line_body,
        grid=(_H,),
        in_specs=[
            pl.BlockSpec(memory_space=pltpu.SMEM),
            pl.BlockSpec(memory_space=pltpu.SMEM),
        ],
        out_specs=pl.BlockSpec((1, _NS, _LINE), lambda h: (h, 0, 0)),
        out_shape=jax.ShapeDtypeStruct((_H, _NS, _LINE), jnp.float32),
    )(scal, relative_attention_bias)

    mesh = plsc.VectorSubcoreMesh(core_axis_name="c", subcore_axis_name="s")
    expand = pl.kernel(
        _sc_expand_body,
        out_type=jax.ShapeDtypeStruct((_H * _Q, _K), jnp.float32),
        mesh=mesh,
        scratch_types=[
            pltpu.VMEM((_NS, _LINE), jnp.float32),
            pltpu.SemaphoreType.DMA,
        ],
        compiler_params=pltpu.CompilerParams(use_tc_tiling_on_sc=False),
    )
    return expand(w16).reshape(1, _H, _Q, _K)


# speedup vs baseline: 2.1847x; 2.1847x over previous
import jax
import jax.numpy as jnp
from jax import lax
from jax.experimental import pallas as pl
from jax.experimental.pallas import tpu as pltpu
from jax.experimental.pallas import tpu_sc as plsc

_NUM_BUCKETS = 32
_H = 16
_Q = 2048
_K = 2048
_LINE = 4224
_SW = 4096
_BI = 128
_HALF = _LINE // 2


def _idx_body(scal_ref, out_ref):
    delta = scal_ref[0]
    boff = scal_ref[1]
    u = jax.lax.broadcasted_iota(jnp.int32, (1, _LINE), 1)
    rel = (2047 - u) + delta
    neg16 = jnp.where(rel < 0, 16, 0)
    n = jnp.abs(rel)
    nf = n.astype(jnp.float32)
    val_large = 8 + (jnp.log(nf / 8.0) / jnp.log(16.0) * 8.0).astype(jnp.int32)
    val_large = jnp.minimum(val_large, 15)
    bucket = neg16 + jnp.where(n < 8, n, val_large) + boff
    out_ref[...] = jnp.mod(bucket, _NUM_BUCKETS)


def _sc_gather_body(idx_hbm, table_hbm, line_hbm, idx_v, tbl_v, line_v):
    c = lax.axis_index("c")
    s = lax.axis_index("s")
    wid = s * 2 + c
    h = wid // 2
    half = wid % 2
    base = half * _HALF
    pltpu.sync_copy(idx_hbm.at[pl.ds(pl.multiple_of(base, 8), _HALF)], idx_v)
    pltpu.sync_copy(table_hbm.at[pl.ds(pl.multiple_of(h * _NUM_BUCKETS, 8),
                                       _NUM_BUCKETS)], tbl_v)

    t0 = tbl_v[pl.ds(0, 16)]
    t1 = tbl_v[pl.ds(16, 16)]

    def g16(vec, iv):
        return lax.gather(
            vec, iv[:, None],
            lax.GatherDimensionNumbers(
                offset_dims=(), collapsed_slice_dims=(0,),
                start_index_map=(0,)),
            slice_sizes=(1,),
            mode=lax.GatherScatterMode.PROMISE_IN_BOUNDS)

    def chunk(k, carry):
        off = pl.multiple_of(k * 16, 16)
        iv = idx_v[pl.ds(off, 16)]
        lo = g16(t0, jnp.minimum(iv, 15))
        hi = g16(t1, jnp.maximum(iv - 16, 0))
        line_v[pl.ds(off, 16)] = jnp.where(iv < 16, lo, hi)
        return carry

    lax.fori_loop(0, _HALF // 16, chunk, 0)
    pltpu.sync_copy(
        line_v,
        line_hbm.at[pl.ds(pl.multiple_of(h * _LINE + base, 8), _HALF)])


def _expand_body(line_ref, out_ref, b_ref, s_ref):
    rb = pl.program_id(1)

    @pl.when(rb == 0)
    def _build_stack():
        line = line_ref[0]
        for r in range(8):
            sh = 7 - r
            row = jnp.concatenate(
                [line[:, sh:], jnp.zeros((1, sh), jnp.float32)], axis=1
            ) if sh else line
            b_ref[pl.ds(r, 1), :] = row
        for q in range(16):
            sh = 120 - 8 * q
            s_ref[pl.ds(8 * q, 8), :] = b_ref[:, sh:sh + _SW]

    o = pl.multiple_of((15 - rb) * _BI, 128)
    out_ref[0, 0] = s_ref[:, pl.ds(o, _K)]


def kernel(q_len, k_len, bidirectional, relative_attention_bias):
    delta = jnp.asarray(q_len, jnp.int32) - jnp.asarray(k_len, jnp.int32)
    boff = jnp.asarray(bidirectional, jnp.int32) - 1
    scal = jnp.stack([delta, boff])

    idx = pl.pallas_call(
        _idx_body,
        in_specs=[pl.BlockSpec(memory_space=pltpu.SMEM)],
        out_shape=jax.ShapeDtypeStruct((1, _LINE), jnp.int32),
    )(scal).reshape(_LINE)

    mesh = plsc.VectorSubcoreMesh(core_axis_name="c", subcore_axis_name="s")
    gather = pl.kernel(
        _sc_gather_body,
        out_type=jax.ShapeDtypeStruct((_H * _LINE,), jnp.float32),
        mesh=mesh,
        scratch_types=[
            pltpu.VMEM((_HALF,), jnp.int32),
            pltpu.VMEM((_NUM_BUCKETS,), jnp.float32),
            pltpu.VMEM((_HALF,), jnp.float32),
        ],
    )
    line_all = gather(idx, relative_attention_bias.reshape(_H * _NUM_BUCKETS))
    line_all = line_all.reshape(_H, 1, _LINE)

    out = pl.pallas_call(
        _expand_body,
        grid=(_H, _Q // _BI),
        in_specs=[pl.BlockSpec((1, 1, _LINE), lambda h, rb: (h, 0, 0))],
        out_specs=pl.BlockSpec((1, 1, _BI, _K), lambda h, rb: (0, h, rb, 0)),
        out_shape=jax.ShapeDtypeStruct((1, _H, _Q, _K), jnp.float32),
        scratch_shapes=[
            pltpu.VMEM((8, _LINE), jnp.float32),
            pltpu.VMEM((_BI, _SW), jnp.float32),
        ],
        compiler_params=pltpu.CompilerParams(
            dimension_semantics=("arbitrary", "arbitrary")),
    )(line_all)
    return out
